# baseline (speedup 1.0000x reference)
import jax
import jax.numpy as jnp
from jax import lax
from jax.experimental import pallas as pl
from jax.experimental.pallas import tpu as pltpu

_ROWS = 32 * 512
_COLS = 512
_BLOCK_ROWS = 1024
_GRID = _ROWS // _BLOCK_ROWS
_STRIP = 32
_TOTAL = float(_ROWS * _COLS)


def _body(reg_ref, tgt_ref, o0, o1, o2, o3, o4, o5, o6, o7, acc_ref):
    i = pl.program_id(0)

    def strip(s, carry):
        af, at, ac = carry
        r = reg_ref[pl.ds(s * _STRIP, _STRIP), :]
        t = tgt_ref[pl.ds(s * _STRIP, _STRIP), :]
        d = r - t
        d2 = d * d
        mf = t > 0.0
        af = af + jnp.where(mf, d2, 0.0)
        at = at + d2
        ac = ac + jnp.where(mf, 1.0, 0.0)
        return af, at, ac

    zero = jnp.zeros((_STRIP, _COLS), jnp.float32)
    af, at, ac = lax.fori_loop(
        0, _BLOCK_ROWS // _STRIP, strip, (zero, zero, zero), unroll=2
    )
    fsum = jnp.sum(af)
    tsum = jnp.sum(at)
    fcnt = jnp.sum(ac)

    @pl.when(i == 0)
    def _():
        acc_ref[0] = fsum
        acc_ref[1] = tsum
        acc_ref[2] = fcnt

    @pl.when(i > 0)
    def _():
        acc_ref[0] += fsum
        acc_ref[1] += tsum
        acc_ref[2] += fcnt

    @pl.when(i == _GRID - 1)
    def _():
        sf = acc_ref[0]
        st = acc_ref[1]
        nf = acc_ref[2]
        su = st - sf
        nu = _TOTAL - nf
        flood = jnp.where(nf > 0.0, sf / jnp.maximum(nf, 1.0), 0.0)
        unflood = jnp.where(nu > 0.0, su / jnp.maximum(nu, 1.0), 0.0)
        loss_reg = 20.0 * flood + unflood
        o0[0] = 2.0 * loss_reg
        o1[0] = 2.0 * loss_reg
        o2[0] = 2.0 * flood
        o3[0] = 2.0 * unflood
        o4[0] = loss_reg
        o5[0] = flood
        o6[0] = unflood
        o7[0] = 0.0


@jax.jit
def _run(reg, targets):
    reg2 = reg.reshape(_ROWS, _COLS)
    tgt2 = targets.reshape(_ROWS, _COLS)
    sds = jax.ShapeDtypeStruct((1,), jnp.float32)
    outs = pl.pallas_call(
        _body,
        grid=(_GRID,),
        in_specs=[
            pl.BlockSpec((_BLOCK_ROWS, _COLS), lambda i: (i, 0)),
            pl.BlockSpec((_BLOCK_ROWS, _COLS), lambda i: (i, 0)),
        ],
        out_specs=[pl.BlockSpec(memory_space=pltpu.SMEM)] * 8,
        out_shape=[sds] * 8,
        scratch_shapes=[pltpu.SMEM((4,), jnp.float32)],
        compiler_params=pltpu.CompilerParams(
            dimension_semantics=("arbitrary",)
        ),
    )(reg2, tgt2)
    return (
        outs[0],
        outs[1].reshape(()),
        outs[2].reshape(()),
        outs[3].reshape(()),
        outs[4].reshape(()),
        outs[5].reshape(()),
        outs[6].reshape(()),
        outs[7],
    )


def kernel(reg, targets):
    return _run(reg, targets)


# speedup vs baseline: 1.1616x; 1.1616x over previous
import jax
import jax.numpy as jnp
from jax import lax
from jax.experimental import pallas as pl
from jax.experimental.pallas import tpu as pltpu

_ROWS = 32 * 512
_COLS = 512
_BLOCK_ROWS = 4096
_GRID = _ROWS // _BLOCK_ROWS
_STRIP = 32
_TOTAL = float(_ROWS * _COLS)


def _body(reg_ref, tgt_ref, o0, o1, o2, o3, o4, o5, o6, o7, acc_ref):
    i = pl.program_id(0)

    def strip(s, carry):
        af, at, ac = carry
        r = reg_ref[pl.ds(s * _STRIP, _STRIP), :]
        t = tgt_ref[pl.ds(s * _STRIP, _STRIP), :]
        d = r - t
        d2 = d * d
        mf = t > 0.0
        af = af + jnp.where(mf, d2, 0.0)
        at = at + d2
        ac = ac + jnp.where(mf, 1.0, 0.0)
        return af, at, ac

    zero = jnp.zeros((_STRIP, _COLS), jnp.float32)
    af, at, ac = lax.fori_loop(
        0, _BLOCK_ROWS // _STRIP, strip, (zero, zero, zero), unroll=2
    )
    fsum = jnp.sum(af)
    tsum = jnp.sum(at)
    fcnt = jnp.sum(ac)

    @pl.when(i == 0)
    def _():
        acc_ref[0] = fsum
        acc_ref[1] = tsum
        acc_ref[2] = fcnt

    @pl.when(i > 0)
    def _():
        acc_ref[0] += fsum
        acc_ref[1] += tsum
        acc_ref[2] += fcnt

    @pl.when(i == _GRID - 1)
    def _():
        sf = acc_ref[0]
        st = acc_ref[1]
        nf = acc_ref[2]
        su = st - sf
        nu = _TOTAL - nf
        flood = jnp.where(nf > 0.0, sf / jnp.maximum(nf, 1.0), 0.0)
        unflood = jnp.where(nu > 0.0, su / jnp.maximum(nu, 1.0), 0.0)
        loss_reg = 20.0 * flood + unflood
        o0[0] = 2.0 * loss_reg
        o1[0] = 2.0 * loss_reg
        o2[0] = 2.0 * flood
        o3[0] = 2.0 * unflood
        o4[0] = loss_reg
        o5[0] = flood
        o6[0] = unflood
        o7[0] = 0.0


@jax.jit
def _run(reg, targets):
    reg2 = reg.reshape(_ROWS, _COLS)
    tgt2 = targets.reshape(_ROWS, _COLS)
    sds = jax.ShapeDtypeStruct((1,), jnp.float32)
    outs = pl.pallas_call(
        _body,
        grid=(_GRID,),
        in_specs=[
            pl.BlockSpec((_BLOCK_ROWS, _COLS), lambda i: (i, 0)),
            pl.BlockSpec((_BLOCK_ROWS, _COLS), lambda i: (i, 0)),
        ],
        out_specs=[pl.BlockSpec(memory_space=pltpu.SMEM)] * 8,
        out_shape=[sds] * 8,
        scratch_shapes=[pltpu.SMEM((4,), jnp.float32)],
        compiler_params=pltpu.CompilerParams(
            dimension_semantics=("arbitrary",)
        ),
    )(reg2, tgt2)
    return (
        outs[0],
        outs[1].reshape(()),
        outs[2].reshape(()),
        outs[3].reshape(()),
        outs[4].reshape(()),
        outs[5].reshape(()),
        outs[6].reshape(()),
        outs[7],
    )


def kernel(reg, targets):
    return _run(reg, targets)


# speedup vs baseline: 1.1690x; 1.0063x over previous
import jax
import jax.numpy as jnp
from jax import lax
from jax.experimental import pallas as pl
from jax.experimental.pallas import tpu as pltpu

_ROWS = 32 * 512
_COLS = 512
_BLOCK_ROWS = 4096
_GRID = _ROWS // _BLOCK_ROWS
_STRIP = 32
_TOTAL = float(_ROWS * _COLS)


def _body(reg_ref, tgt_ref, o0, o1, o2, o3, o4, o5, o6, o7, acc_ref):
    i = pl.program_id(0)

    r = reg_ref[...]
    t = tgt_ref[...]
    d = r - t
    d2 = d * d
    mf = t > 0.0
    md2 = jnp.where(mf, d2, 0.0)
    fsum = jnp.sum(md2)
    tsum = jnp.sum(d2)
    fcnt = jnp.sum(jnp.where(mf, 1.0, 0.0))

    @pl.when(i == 0)
    def _():
        acc_ref[0] = fsum
        acc_ref[1] = tsum
        acc_ref[2] = fcnt

    @pl.when(i > 0)
    def _():
        acc_ref[0] += fsum
        acc_ref[1] += tsum
        acc_ref[2] += fcnt

    @pl.when(i == _GRID - 1)
    def _():
        sf = acc_ref[0]
        st = acc_ref[1]
        nf = acc_ref[2]
        su = st - sf
        nu = _TOTAL - nf
        flood = jnp.where(nf > 0.0, sf / jnp.maximum(nf, 1.0), 0.0)
        unflood = jnp.where(nu > 0.0, su / jnp.maximum(nu, 1.0), 0.0)
        loss_reg = 20.0 * flood + unflood
        o0[0] = 2.0 * loss_reg
        o1[0] = 2.0 * loss_reg
        o2[0] = 2.0 * flood
        o3[0] = 2.0 * unflood
        o4[0] = loss_reg
        o5[0] = flood
        o6[0] = unflood
        o7[0] = 0.0


@jax.jit
def _run(reg, targets):
    reg2 = reg.reshape(_ROWS, _COLS)
    tgt2 = targets.reshape(_ROWS, _COLS)
    sds = jax.ShapeDtypeStruct((1,), jnp.float32)
    outs = pl.pallas_call(
        _body,
        grid=(_GRID,),
        in_specs=[
            pl.BlockSpec((_BLOCK_ROWS, _COLS), lambda i: (i, 0)),
            pl.BlockSpec((_BLOCK_ROWS, _COLS), lambda i: (i, 0)),
        ],
        out_specs=[pl.BlockSpec(memory_space=pltpu.SMEM)] * 8,
        out_shape=[sds] * 8,
        scratch_shapes=[pltpu.SMEM((4,), jnp.float32)],
        compiler_params=pltpu.CompilerParams(
            dimension_semantics=("arbitrary",)
        ),
    )(reg2, tgt2)
    return (
        outs[0],
        outs[1].reshape(()),
        outs[2].reshape(()),
        outs[3].reshape(()),
        outs[4].reshape(()),
        outs[5].reshape(()),
        outs[6].reshape(()),
        outs[7],
    )


def kernel(reg, targets):
    return _run(reg, targets)


# speedup vs baseline: 1.1797x; 1.0092x over previous
import jax
import jax.numpy as jnp
from jax import lax
from jax.experimental import pallas as pl
from jax.experimental.pallas import tpu as pltpu

_ROWS = 32 * 512
_COLS = 512
_BLOCK_ROWS = 2048
_GRID = _ROWS // _BLOCK_ROWS // 2
_STRIP = 32
_TOTAL = float(_ROWS * _COLS)


def _half_sums(r, t):
    d = r - t
    d2 = d * d
    mf = t > 0.0
    md2 = jnp.where(mf, d2, 0.0)
    return (
        jnp.sum(md2),
        jnp.sum(d2),
        jnp.sum(jnp.where(mf, 1.0, 0.0)),
    )


def _body(r1_ref, r2_ref, t1_ref, t2_ref, o0, o1, o2, o3, o4, o5, o6, o7,
          acc_ref):
    i = pl.program_id(0)

    f1, s1, c1 = _half_sums(r1_ref[...], t1_ref[...])
    f2, s2, c2 = _half_sums(r2_ref[...], t2_ref[...])
    fsum = f1 + f2
    tsum = s1 + s2
    fcnt = c1 + c2

    @pl.when(i == 0)
    def _():
        acc_ref[0] = fsum
        acc_ref[1] = tsum
        acc_ref[2] = fcnt

    @pl.when(i > 0)
    def _():
        acc_ref[0] += fsum
        acc_ref[1] += tsum
        acc_ref[2] += fcnt

    @pl.when(i == _GRID - 1)
    def _():
        sf = acc_ref[0]
        st = acc_ref[1]
        nf = acc_ref[2]
        su = st - sf
        nu = _TOTAL - nf
        flood = jnp.where(nf > 0.0, sf / jnp.maximum(nf, 1.0), 0.0)
        unflood = jnp.where(nu > 0.0, su / jnp.maximum(nu, 1.0), 0.0)
        loss_reg = 20.0 * flood + unflood
        o0[0] = 2.0 * loss_reg
        o1[0] = 2.0 * loss_reg
        o2[0] = 2.0 * flood
        o3[0] = 2.0 * unflood
        o4[0] = loss_reg
        o5[0] = flood
        o6[0] = unflood
        o7[0] = 0.0


@jax.jit
def _run(reg, targets):
    reg2 = reg.reshape(_ROWS, _COLS)
    tgt2 = targets.reshape(_ROWS, _COLS)
    sds = jax.ShapeDtypeStruct((1,), jnp.float32)
    outs = pl.pallas_call(
        _body,
        grid=(_GRID,),
        in_specs=[
            pl.BlockSpec((_BLOCK_ROWS, _COLS), lambda i: (i, 0)),
            pl.BlockSpec((_BLOCK_ROWS, _COLS), lambda i: (i + _GRID, 0)),
            pl.BlockSpec((_BLOCK_ROWS, _COLS), lambda i: (i, 0)),
            pl.BlockSpec((_BLOCK_ROWS, _COLS), lambda i: (i + _GRID, 0)),
        ],
        out_specs=[pl.BlockSpec(memory_space=pltpu.SMEM)] * 8,
        out_shape=[sds] * 8,
        scratch_shapes=[pltpu.SMEM((4,), jnp.float32)],
        compiler_params=pltpu.CompilerParams(
            dimension_semantics=("arbitrary",)
        ),
    )(reg2, reg2, tgt2, tgt2)
    return (
        outs[0],
        outs[1].reshape(()),
        outs[2].reshape(()),
        outs[3].reshape(()),
        outs[4].reshape(()),
        outs[5].reshape(()),
        outs[6].reshape(()),
        outs[7],
    )


def kernel(reg, targets):
    return _run(reg, targets)
